# Initial kernel scaffold; baseline (speedup 1.0000x reference)
import jax
import jax.numpy as jnp
from jax.experimental import pallas as pl


def kernel(b_coords, b_o, b_scores):
    raise NotImplementedError("write your pallas kernel here")


# speedup vs baseline: 16.8371x; 16.8371x over previous
import jax
import jax.numpy as jnp
from jax.experimental import pallas as pl
from jax.experimental.pallas import tpu as pltpu

_C = 20
_NP = 5120
_T = 512
_NT = _NP // _T
_IOU_T = 0.7
_SCORE_T = 0.05
_NEG = -jnp.inf

_QX1, _QY1, _QX2, _QY2, _QAR, _QCF, _QID, _QVL, _QCE, _QKP = range(10)


def _iou_tile(x1s, y1s, x2s, y2s, ars, x1t, y1t, x2t, y2t, art):
    ix1 = jnp.maximum(x1s, x1t)
    iy1 = jnp.maximum(y1s, y1t)
    ix2 = jnp.minimum(x2s, x2t)
    iy2 = jnp.minimum(y2s, y2t)
    iw = jnp.maximum(ix2 - ix1, 0.0)
    ih = jnp.maximum(iy2 - iy1, 0.0)
    inter = iw * ih
    union = ars + art - inter
    return inter / jnp.maximum(union, 1e-9)


def _quantities(x, y, w, h, o, m):
    x1 = jnp.clip(x - w / 2.0, 0.0, 1.0)
    y1 = jnp.clip(y - h / 2.0, 0.0, 1.0)
    x2 = jnp.clip(x + w / 2.0, 0.0, 1.0)
    y2 = jnp.clip(y + h / 2.0, 0.0, 1.0)
    area = (x2 - x1) * (y2 - y1)
    conf = m * o
    valid = jnp.where((o >= 0.5) & (conf >= _SCORE_T), 1.0, 0.0)
    return x1, y1, x2, y2, area, conf, valid


def _nms_body(rp_ref, sr_ref, cp_ref, sc_ref, out_ref, qrow, qcol):
    f32 = jnp.float32

    rp = rp_ref[0]
    s_r = sr_ref[0]
    m_r = jnp.max(s_r, axis=1, keepdims=True)
    x1, y1, x2, y2, area, conf_r, valid_r = _quantities(
        rp[:, 0:1], rp[:, 1:2], rp[:, 2:3], rp[:, 3:4], rp[:, 4:5], m_r)
    idx_r = jax.lax.broadcasted_iota(jnp.int32, (_NP, 1), 0).astype(f32)
    qrow[:, _QX1:_QX1 + 1] = x1
    qrow[:, _QY1:_QY1 + 1] = y1
    qrow[:, _QX2:_QX2 + 1] = x2
    qrow[:, _QY2:_QY2 + 1] = y2
    qrow[:, _QAR:_QAR + 1] = area
    qrow[:, _QCF:_QCF + 1] = conf_r
    qrow[:, _QID:_QID + 1] = idx_r
    qrow[:, _QVL:_QVL + 1] = valid_r
    qrow[:, _QKP:_QKP + 1] = valid_r

    cp = cp_ref[0]
    s_c = sc_ref[0]
    m_c = jnp.max(s_c, axis=0, keepdims=True)
    x1c, y1c, x2c, y2c, areac, conf_c, valid_c = _quantities(
        cp[0:1, :], cp[1:2, :], cp[2:3, :], cp[3:4, :], cp[4:5, :], m_c)
    idx_c = jax.lax.broadcasted_iota(jnp.int32, (1, _NP), 1).astype(f32)
    qcol[_QX1:_QX1 + 1, :] = x1c
    qcol[_QY1:_QY1 + 1, :] = y1c
    qcol[_QX2:_QX2 + 1, :] = x2c
    qcol[_QY2:_QY2 + 1, :] = y2c
    qcol[_QAR:_QAR + 1, :] = areac
    qcol[_QCF:_QCF + 1, :] = conf_c
    qcol[_QID:_QID + 1, :] = idx_c
    qcol[_QVL:_QVL + 1, :] = valid_c

    def sweep_pair(changed_unused):
        qrow[:, _QCE:_QCE + 1] = jnp.where(
            qrow[:, _QKP:_QKP + 1] > 0.0, qrow[:, _QCF:_QCF + 1], _NEG)

        def jbody(j, _):
            off = pl.multiple_of(j * _T, _T)
            x1t = qcol[_QX1:_QX1 + 1, pl.ds(off, _T)]
            y1t = qcol[_QY1:_QY1 + 1, pl.ds(off, _T)]
            x2t = qcol[_QX2:_QX2 + 1, pl.ds(off, _T)]
            y2t = qcol[_QY2:_QY2 + 1, pl.ds(off, _T)]
            art = qcol[_QAR:_QAR + 1, pl.ds(off, _T)]
            ct = qcol[_QCF:_QCF + 1, pl.ds(off, _T)]
            it = qcol[_QID:_QID + 1, pl.ds(off, _T)]
            vt = qcol[_QVL:_QVL + 1, pl.ds(off, _T)]

            def ibody(i, acc):
                roff = pl.multiple_of(i * _T, _T)
                x1s = qrow[pl.ds(roff, _T), _QX1:_QX1 + 1]
                y1s = qrow[pl.ds(roff, _T), _QY1:_QY1 + 1]
                x2s = qrow[pl.ds(roff, _T), _QX2:_QX2 + 1]
                y2s = qrow[pl.ds(roff, _T), _QY2:_QY2 + 1]
                ars = qrow[pl.ds(roff, _T), _QAR:_QAR + 1]
                iss = qrow[pl.ds(roff, _T), _QID:_QID + 1]
                ces = qrow[pl.ds(roff, _T), _QCE:_QCE + 1]
                iou = _iou_tile(x1s, y1s, x2s, y2s, ars, x1t, y1t, x2t, y2t, art)
                sup = (iou > _IOU_T) & ((ces > ct) | ((ces == ct) & (iss < it)))
                return jnp.maximum(
                    acc, jnp.max(jnp.where(sup, 1.0, 0.0), axis=0, keepdims=True))

            acc = jax.lax.fori_loop(0, _NT, ibody, jnp.zeros((1, _T), f32))
            qcol[_QKP:_QKP + 1, pl.ds(off, _T)] = vt * (1.0 - acc)
            return 0

        jax.lax.fori_loop(0, _NT, jbody, 0)

        qcol[_QCE:_QCE + 1, :] = jnp.where(
            qcol[_QKP:_QKP + 1, :] > 0.0, qcol[_QCF:_QCF + 1, :], _NEG)

        def ibody2(i, changed):
            roff = pl.multiple_of(i * _T, _T)
            x1t = qrow[pl.ds(roff, _T), _QX1:_QX1 + 1]
            y1t = qrow[pl.ds(roff, _T), _QY1:_QY1 + 1]
            x2t = qrow[pl.ds(roff, _T), _QX2:_QX2 + 1]
            y2t = qrow[pl.ds(roff, _T), _QY2:_QY2 + 1]
            art = qrow[pl.ds(roff, _T), _QAR:_QAR + 1]
            ct = qrow[pl.ds(roff, _T), _QCF:_QCF + 1]
            it = qrow[pl.ds(roff, _T), _QID:_QID + 1]
            vt = qrow[pl.ds(roff, _T), _QVL:_QVL + 1]

            def jbody2(j, acc):
                off = pl.multiple_of(j * _T, _T)
                x1s = qcol[_QX1:_QX1 + 1, pl.ds(off, _T)]
                y1s = qcol[_QY1:_QY1 + 1, pl.ds(off, _T)]
                x2s = qcol[_QX2:_QX2 + 1, pl.ds(off, _T)]
                y2s = qcol[_QY2:_QY2 + 1, pl.ds(off, _T)]
                ars = qcol[_QAR:_QAR + 1, pl.ds(off, _T)]
                iss = qcol[_QID:_QID + 1, pl.ds(off, _T)]
                ces = qcol[_QCE:_QCE + 1, pl.ds(off, _T)]
                iou = _iou_tile(x1s, y1s, x2s, y2s, ars, x1t, y1t, x2t, y2t, art)
                sup = (iou > _IOU_T) & ((ces > ct) | ((ces == ct) & (iss < it)))
                return jnp.maximum(
                    acc, jnp.max(jnp.where(sup, 1.0, 0.0), axis=1, keepdims=True))

            acc = jax.lax.fori_loop(0, _NT, jbody2, jnp.zeros((_T, 1), f32))
            knew = vt * (1.0 - acc)
            old = qrow[pl.ds(roff, _T), _QKP:_QKP + 1]
            changed = changed | jnp.any(knew != old)
            qrow[pl.ds(roff, _T), _QKP:_QKP + 1] = knew
            return changed

        return jax.lax.fori_loop(0, _NT, ibody2, jnp.bool_(False))

    jax.lax.while_loop(lambda c: c, sweep_pair, jnp.bool_(True))

    kf = qrow[:, _QKP:_QKP + 1]
    out_ref[0, :, 0:4] = qrow[:, _QX1:_QY2 + 1] * kf
    out_ref[0, :, 4:5] = qrow[:, _QCF:_QCF + 1] * kf
    lbl_iota = jax.lax.broadcasted_iota(jnp.int32, (_NP, _C), 1).astype(f32)
    lbl = jnp.min(jnp.where(s_r == m_r, lbl_iota, float(_C)), axis=1, keepdims=True)
    out_ref[0, :, 5:6] = jnp.where(kf > 0.0, lbl, -1.0)
    out_ref[0, :, 6:7] = kf


def _run(rowpack, scores_p, interpret=False):
    B = rowpack.shape[0]
    f32 = jnp.float32
    colpack = rowpack.transpose(0, 2, 1)
    sc = scores_p.transpose(0, 2, 1)

    spec = lambda s: pl.BlockSpec(s, lambda b: (b,) + (0,) * (len(s) - 1))
    return pl.pallas_call(
        _nms_body,
        grid=(B,),
        in_specs=[
            spec((1, _NP, 5)),
            spec((1, _NP, _C)),
            spec((1, 5, _NP)),
            spec((1, _C, _NP)),
        ],
        out_specs=spec((1, _NP, 8)),
        out_shape=jax.ShapeDtypeStruct((B, _NP, 8), f32),
        scratch_shapes=[
            pltpu.VMEM((_NP, 16), f32),
            pltpu.VMEM((16, _NP), f32),
        ],
        interpret=interpret,
    )(rowpack, scores_p, colpack, sc)


def kernel(b_coords, b_o, b_scores):
    B, N, _ = b_coords.shape
    pad = _NP - N
    coords_p = jnp.pad(b_coords, ((0, 0), (0, pad), (0, 0)))
    o_p = jnp.pad(b_o, ((0, 0), (0, pad)))
    scores_p = jnp.pad(b_scores, ((0, 0), (0, pad), (0, 0)))
    rowpack = jnp.concatenate([coords_p, o_p[..., None]], axis=2)
    out = _run(rowpack, scores_p)
    boxes = out[:, :N, 0:4]
    scores = out[:, :N, 4]
    labels = out[:, :N, 5].astype(jnp.int32)
    keep = out[:, :N, 6].astype(jnp.bool_)
    return boxes, scores, labels, keep


# speedup vs baseline: 139.2134x; 8.2682x over previous
import jax
import jax.numpy as jnp
from jax.experimental import pallas as pl
from jax.experimental.pallas import tpu as pltpu

_C = 20
_NP = 5120
_T = 512
_NT = _NP // _T
_PK = 16
_TP = _T // _PK
_NPP = _NP // _PK
_IOU_T = 0.7
_SCORE_T = 0.05

_QX1, _QY1, _QX2, _QY2, _QAR, _QCF, _QID, _QVL, _QKP = range(9)


def _iou_tile(x1s, y1s, x2s, y2s, ars, x1t, y1t, x2t, y2t, art):
    ix1 = jnp.maximum(x1s, x1t)
    iy1 = jnp.maximum(y1s, y1t)
    ix2 = jnp.minimum(x2s, x2t)
    iy2 = jnp.minimum(y2s, y2t)
    iw = jnp.maximum(ix2 - ix1, 0.0)
    ih = jnp.maximum(iy2 - iy1, 0.0)
    inter = iw * ih
    union = ars + art - inter
    return inter / jnp.maximum(union, 1e-9)


def _quantities(x, y, w, h, o, m):
    x1 = jnp.clip(x - w / 2.0, 0.0, 1.0)
    y1 = jnp.clip(y - h / 2.0, 0.0, 1.0)
    x2 = jnp.clip(x + w / 2.0, 0.0, 1.0)
    y2 = jnp.clip(y + h / 2.0, 0.0, 1.0)
    area = (x2 - x1) * (y2 - y1)
    conf = m * o
    valid = jnp.where((o >= 0.5) & (conf >= _SCORE_T), 1.0, 0.0)
    return x1, y1, x2, y2, area, conf, valid


def _pow2_f32(e):
    return jax.lax.bitcast_convert_type((e + 127) << 23, jnp.float32)


def _dot(a, b):
    return jax.lax.dot_general(a, b, (((1,), (0,)), ((), ())),
                               preferred_element_type=jnp.float32)


def _dot_rt(a, b):
    return jax.lax.dot_general(a, b, (((1,), (1,)), ((), ())),
                               preferred_element_type=jnp.float32)


def _nms_body(rp_ref, sr_ref, cp_ref, sc_ref, out_ref, qrow, qcol, mpa, kpr):
    f32 = jnp.float32
    i32 = jnp.int32

    c_l = jax.lax.broadcasted_iota(i32, (_TP, _T), 1)
    r_s = jax.lax.broadcasted_iota(i32, (_TP, _T), 0)
    pmat = jnp.where((c_l >> 4) == r_s, _pow2_f32(c_l & 15), 0.0)

    rp = rp_ref[0]
    s_r = sr_ref[0]
    m_r = jnp.max(s_r, axis=1, keepdims=True)
    x1, y1, x2, y2, area, conf_r, valid_r = _quantities(
        rp[:, 0:1], rp[:, 1:2], rp[:, 2:3], rp[:, 3:4], rp[:, 4:5], m_r)
    idx_r = jax.lax.broadcasted_iota(i32, (_NP, 1), 0).astype(f32)
    qrow[:, _QX1:_QX1 + 1] = x1
    qrow[:, _QY1:_QY1 + 1] = y1
    qrow[:, _QX2:_QX2 + 1] = x2
    qrow[:, _QY2:_QY2 + 1] = y2
    qrow[:, _QAR:_QAR + 1] = area
    qrow[:, _QCF:_QCF + 1] = conf_r
    qrow[:, _QID:_QID + 1] = idx_r
    qrow[:, _QVL:_QVL + 1] = valid_r

    cp = cp_ref[0]
    s_c = sc_ref[0]
    m_c = jnp.max(s_c, axis=0, keepdims=True)
    x1c, y1c, x2c, y2c, areac, conf_c, valid_c = _quantities(
        cp[0:1, :], cp[1:2, :], cp[2:3, :], cp[3:4, :], cp[4:5, :], m_c)
    idx_c = jax.lax.broadcasted_iota(i32, (1, _NP), 1).astype(f32)
    qcol[_QX1:_QX1 + 1, :] = x1c
    qcol[_QY1:_QY1 + 1, :] = y1c
    qcol[_QX2:_QX2 + 1, :] = x2c
    qcol[_QY2:_QY2 + 1, :] = y2c
    qcol[_QAR:_QAR + 1, :] = areac
    qcol[_QCF:_QCF + 1, :] = conf_c
    qcol[_QID:_QID + 1, :] = idx_c
    qcol[_QVL:_QVL + 1, :] = valid_c
    qcol[_QKP:_QKP + 1, :] = valid_c

    def build_i(i, _):
        roff = pl.multiple_of(i * _T, _T)
        x1s = qrow[pl.ds(roff, _T), _QX1:_QX1 + 1]
        y1s = qrow[pl.ds(roff, _T), _QY1:_QY1 + 1]
        x2s = qrow[pl.ds(roff, _T), _QX2:_QX2 + 1]
        y2s = qrow[pl.ds(roff, _T), _QY2:_QY2 + 1]
        ars = qrow[pl.ds(roff, _T), _QAR:_QAR + 1]
        cfs = qrow[pl.ds(roff, _T), _QCF:_QCF + 1]
        ids = qrow[pl.ds(roff, _T), _QID:_QID + 1]
        poff = pl.multiple_of(i * _TP, _TP)

        def build_j(j, _):
            off = pl.multiple_of(j * _T, _T)
            x1t = qcol[_QX1:_QX1 + 1, pl.ds(off, _T)]
            y1t = qcol[_QY1:_QY1 + 1, pl.ds(off, _T)]
            x2t = qcol[_QX2:_QX2 + 1, pl.ds(off, _T)]
            y2t = qcol[_QY2:_QY2 + 1, pl.ds(off, _T)]
            art = qcol[_QAR:_QAR + 1, pl.ds(off, _T)]
            cft = qcol[_QCF:_QCF + 1, pl.ds(off, _T)]
            idt = qcol[_QID:_QID + 1, pl.ds(off, _T)]
            iou = _iou_tile(x1s, y1s, x2s, y2s, ars, x1t, y1t, x2t, y2t, art)
            sup = (iou > _IOU_T) & ((cfs > cft) | ((cfs == cft) & (ids < idt)))
            t_a = jnp.where(sup, 1.0, 0.0)
            mpa[pl.ds(poff, _TP), pl.ds(off, _T)] = _dot(pmat, t_a).astype(i32)
            return 0

        jax.lax.fori_loop(0, _NT, build_j, 0)
        return 0

    jax.lax.fori_loop(0, _NT, build_i, 0)

    def pack_i(i, _):
        off = pl.multiple_of(i * _T, _T)
        kpr[pl.ds(i * _TP, _TP), 0:1] = _dot_rt(
            pmat, qcol[_QVL:_QVL + 1, pl.ds(off, _T)]).astype(i32)
        return 0

    jax.lax.fori_loop(0, _NT, pack_i, 0)

    def sweep(changed_unused):
        def jbody(j, changed):
            off = pl.multiple_of(j * _T, _T)
            hit = (mpa[:, pl.ds(off, _T)] & kpr[:, 0:1]) != 0
            sup = jnp.max(jnp.where(hit, 1.0, 0.0), axis=0, keepdims=True)
            vt = qcol[_QVL:_QVL + 1, pl.ds(off, _T)]
            knew = vt * (1.0 - sup)
            old = qcol[_QKP:_QKP + 1, pl.ds(off, _T)]
            changed = changed | jnp.any(knew != old)
            qcol[_QKP:_QKP + 1, pl.ds(off, _T)] = knew
            kpr[pl.ds(j * _TP, _TP), 0:1] = _dot_rt(pmat, knew).astype(jnp.int32)
            return changed

        return jax.lax.fori_loop(0, _NT, jbody, jnp.bool_(False))

    jax.lax.while_loop(lambda c: c, sweep, jnp.bool_(True))

    eye_r = jax.lax.broadcasted_iota(i32, (_T, _T), 0)
    eye_c = jax.lax.broadcasted_iota(i32, (_T, _T), 1)
    eye = jnp.where(eye_r == eye_c, 1.0, 0.0)

    def unpack_i(i, _):
        off = pl.multiple_of(i * _T, _T)
        qrow[pl.ds(off, _T), _QKP:_QKP + 1] = _dot_rt(
            eye, qcol[_QKP:_QKP + 1, pl.ds(off, _T)])
        return 0

    jax.lax.fori_loop(0, _NT, unpack_i, 0)

    kf = qrow[:, _QKP:_QKP + 1]
    out_ref[0, :, 0:4] = qrow[:, _QX1:_QY2 + 1] * kf
    out_ref[0, :, 4:5] = qrow[:, _QCF:_QCF + 1] * kf
    lbl_iota = jax.lax.broadcasted_iota(jnp.int32, (_NP, _C), 1).astype(f32)
    lbl = jnp.min(jnp.where(s_r == m_r, lbl_iota, float(_C)), axis=1, keepdims=True)
    out_ref[0, :, 5:6] = jnp.where(kf > 0.0, lbl, -1.0)
    out_ref[0, :, 6:7] = kf


def _run(rowpack, scores_p, interpret=False):
    B = rowpack.shape[0]
    f32 = jnp.float32
    colpack = rowpack.transpose(0, 2, 1)
    sc = scores_p.transpose(0, 2, 1)

    spec = lambda s: pl.BlockSpec(s, lambda b: (b,) + (0,) * (len(s) - 1))
    return pl.pallas_call(
        _nms_body,
        grid=(B,),
        in_specs=[
            spec((1, _NP, 5)),
            spec((1, _NP, _C)),
            spec((1, 5, _NP)),
            spec((1, _C, _NP)),
        ],
        out_specs=spec((1, _NP, 8)),
        out_shape=jax.ShapeDtypeStruct((B, _NP, 8), f32),
        scratch_shapes=[
            pltpu.VMEM((_NP, 16), f32),
            pltpu.VMEM((16, _NP), f32),
            pltpu.VMEM((_NPP, _NP), jnp.int32),
            pltpu.VMEM((_NPP, 1), jnp.int32),
        ],
        interpret=interpret,
    )(rowpack, scores_p, colpack, sc)


def kernel(b_coords, b_o, b_scores):
    B, N, _ = b_coords.shape
    pad = _NP - N
    coords_p = jnp.pad(b_coords, ((0, 0), (0, pad), (0, 0)))
    o_p = jnp.pad(b_o, ((0, 0), (0, pad)))
    scores_p = jnp.pad(b_scores, ((0, 0), (0, pad), (0, 0)))
    rowpack = jnp.concatenate([coords_p, o_p[..., None]], axis=2)
    out = _run(rowpack, scores_p)
    boxes = out[:, :N, 0:4]
    scores = out[:, :N, 4]
    labels = out[:, :N, 5].astype(jnp.int32)
    keep = out[:, :N, 6].astype(jnp.bool_)
    return boxes, scores, labels, keep


# speedup vs baseline: 218.2801x; 1.5680x over previous
import jax
import jax.numpy as jnp
from jax.experimental import pallas as pl
from jax.experimental.pallas import tpu as pltpu

_C = 20
_NP = 5120
_T = 512
_NT = _NP // _T
_PK = 16
_TP = _T // _PK
_NPP = _NP // _PK
_IOU_T = 0.7
_SCORE_T = 0.05

_QX1, _QY1, _QX2, _QY2, _QAR, _QCF, _QID, _QVL, _QKP = range(9)


def _iou_tile(x1s, y1s, x2s, y2s, ars, x1t, y1t, x2t, y2t, art):
    ix1 = jnp.maximum(x1s, x1t)
    iy1 = jnp.maximum(y1s, y1t)
    ix2 = jnp.minimum(x2s, x2t)
    iy2 = jnp.minimum(y2s, y2t)
    iw = jnp.maximum(ix2 - ix1, 0.0)
    ih = jnp.maximum(iy2 - iy1, 0.0)
    inter = iw * ih
    union = ars + art - inter
    return inter / jnp.maximum(union, 1e-9)


def _quantities(x, y, w, h, o, m):
    x1 = jnp.clip(x - w / 2.0, 0.0, 1.0)
    y1 = jnp.clip(y - h / 2.0, 0.0, 1.0)
    x2 = jnp.clip(x + w / 2.0, 0.0, 1.0)
    y2 = jnp.clip(y + h / 2.0, 0.0, 1.0)
    area = (x2 - x1) * (y2 - y1)
    conf = m * o
    valid = jnp.where((o >= 0.5) & (conf >= _SCORE_T), 1.0, 0.0)
    return x1, y1, x2, y2, area, conf, valid


def _pow2_f32(e):
    return jax.lax.bitcast_convert_type((e + 127) << 23, jnp.float32)


def _dot(a, b):
    return jax.lax.dot_general(a, b, (((1,), (0,)), ((), ())),
                               preferred_element_type=jnp.float32)


def _dot_rt(a, b):
    return jax.lax.dot_general(a, b, (((1,), (1,)), ((), ())),
                               preferred_element_type=jnp.float32)


def _nms_body(rp_ref, sr_ref, cp_ref, sc_ref, out_ref, qrow, qcol, mpa, kpr):
    f32 = jnp.float32
    i32 = jnp.int32

    c_l = jax.lax.broadcasted_iota(i32, (_TP, _T), 1)
    r_s = jax.lax.broadcasted_iota(i32, (_TP, _T), 0)
    pmat = jnp.where((c_l >> 4) == r_s, _pow2_f32(c_l & 15), 0.0)

    rp = rp_ref[0]
    s_r = sr_ref[0]
    m_r = jnp.max(s_r, axis=1, keepdims=True)
    x1, y1, x2, y2, area, conf_r, valid_r = _quantities(
        rp[:, 0:1], rp[:, 1:2], rp[:, 2:3], rp[:, 3:4], rp[:, 4:5], m_r)
    idx_r = jax.lax.broadcasted_iota(i32, (_NP, 1), 0).astype(f32)
    qrow[:, _QX1:_QX1 + 1] = x1
    qrow[:, _QY1:_QY1 + 1] = y1
    qrow[:, _QX2:_QX2 + 1] = x2
    qrow[:, _QY2:_QY2 + 1] = y2
    qrow[:, _QAR:_QAR + 1] = area
    qrow[:, _QCF:_QCF + 1] = conf_r
    qrow[:, _QID:_QID + 1] = idx_r
    qrow[:, _QVL:_QVL + 1] = valid_r

    cp = cp_ref[0]
    s_c = sc_ref[0]
    m_c = jnp.max(s_c, axis=0, keepdims=True)
    x1c, y1c, x2c, y2c, areac, conf_c, valid_c = _quantities(
        cp[0:1, :], cp[1:2, :], cp[2:3, :], cp[3:4, :], cp[4:5, :], m_c)
    idx_c = jax.lax.broadcasted_iota(i32, (1, _NP), 1).astype(f32)
    qcol[_QX1:_QX1 + 1, :] = x1c
    qcol[_QY1:_QY1 + 1, :] = y1c
    qcol[_QX2:_QX2 + 1, :] = x2c
    qcol[_QY2:_QY2 + 1, :] = y2c
    qcol[_QAR:_QAR + 1, :] = areac
    qcol[_QCF:_QCF + 1, :] = conf_c
    qcol[_QID:_QID + 1, :] = idx_c
    qcol[_QVL:_QVL + 1, :] = valid_c
    qcol[_QKP:_QKP + 1, :] = valid_c

    def col_q(off):
        return (qcol[_QX1:_QX1 + 1, pl.ds(off, _T)],
                qcol[_QY1:_QY1 + 1, pl.ds(off, _T)],
                qcol[_QX2:_QX2 + 1, pl.ds(off, _T)],
                qcol[_QY2:_QY2 + 1, pl.ds(off, _T)],
                qcol[_QAR:_QAR + 1, pl.ds(off, _T)],
                qcol[_QCF:_QCF + 1, pl.ds(off, _T)],
                qcol[_QID:_QID + 1, pl.ds(off, _T)])

    def build_i(i, _):
        roff = pl.multiple_of(i * _T, _T)
        x1s = qrow[pl.ds(roff, _T), _QX1:_QX1 + 1]
        y1s = qrow[pl.ds(roff, _T), _QY1:_QY1 + 1]
        x2s = qrow[pl.ds(roff, _T), _QX2:_QX2 + 1]
        y2s = qrow[pl.ds(roff, _T), _QY2:_QY2 + 1]
        ars = qrow[pl.ds(roff, _T), _QAR:_QAR + 1]
        cfs = qrow[pl.ds(roff, _T), _QCF:_QCF + 1]
        ids = qrow[pl.ds(roff, _T), _QID:_QID + 1]
        poff = pl.multiple_of(i * _TP, _TP)

        x1t, y1t, x2t, y2t, art, cft, idt = col_q(roff)
        iou = _iou_tile(x1s, y1s, x2s, y2s, ars, x1t, y1t, x2t, y2t, art)
        s_b = iou > _IOU_T
        h_f = jnp.where((cfs > cft) | ((cfs == cft) & (ids < idt)), 1.0, 0.0)
        t_a = jnp.where(s_b, h_f, 0.0)
        mpa[pl.ds(poff, _TP), pl.ds(roff, _T)] = _dot(pmat, t_a).astype(i32)

        def build_j(j, _):
            off = pl.multiple_of(j * _T, _T)
            x1t, y1t, x2t, y2t, art, cft, idt = col_q(off)
            iou = _iou_tile(x1s, y1s, x2s, y2s, ars, x1t, y1t, x2t, y2t, art)
            s_b = iou > _IOU_T
            h_f = jnp.where((cfs > cft) | ((cfs == cft) & (ids < idt)), 1.0, 0.0)
            t_a = jnp.where(s_b, h_f, 0.0)
            t_b = jnp.where(s_b, 1.0 - h_f, 0.0)
            mpa[pl.ds(poff, _TP), pl.ds(off, _T)] = _dot(pmat, t_a).astype(i32)
            mpa[pl.ds(j * _TP, _TP), pl.ds(roff, _T)] = _dot_rt(
                pmat, t_b).astype(i32)
            return 0

        jax.lax.fori_loop(i + 1, _NT, build_j, 0)
        return 0

    jax.lax.fori_loop(0, _NT, build_i, 0)

    def pack_i(i, _):
        off = pl.multiple_of(i * _T, _T)
        kpr[pl.ds(i * _TP, _TP), 0:1] = _dot_rt(
            pmat, qcol[_QVL:_QVL + 1, pl.ds(off, _T)]).astype(i32)
        return 0

    jax.lax.fori_loop(0, _NT, pack_i, 0)

    def sweep(changed_unused):
        def jbody(j, changed):
            off = pl.multiple_of(j * _T, _T)
            hit = (mpa[:, pl.ds(off, _T)] & kpr[:, 0:1]) != 0
            sup = jnp.max(jnp.where(hit, 1.0, 0.0), axis=0, keepdims=True)
            vt = qcol[_QVL:_QVL + 1, pl.ds(off, _T)]
            knew = vt * (1.0 - sup)
            old = qcol[_QKP:_QKP + 1, pl.ds(off, _T)]
            changed = changed | jnp.any(knew != old)
            qcol[_QKP:_QKP + 1, pl.ds(off, _T)] = knew
            kpr[pl.ds(j * _TP, _TP), 0:1] = _dot_rt(pmat, knew).astype(jnp.int32)
            return changed

        return jax.lax.fori_loop(0, _NT, jbody, jnp.bool_(False))

    jax.lax.while_loop(lambda c: c, sweep, jnp.bool_(True))

    eye_r = jax.lax.broadcasted_iota(i32, (_T, _T), 0)
    eye_c = jax.lax.broadcasted_iota(i32, (_T, _T), 1)
    eye = jnp.where(eye_r == eye_c, 1.0, 0.0)

    def unpack_i(i, _):
        off = pl.multiple_of(i * _T, _T)
        qrow[pl.ds(off, _T), _QKP:_QKP + 1] = _dot_rt(
            eye, qcol[_QKP:_QKP + 1, pl.ds(off, _T)])
        return 0

    jax.lax.fori_loop(0, _NT, unpack_i, 0)

    kf = qrow[:, _QKP:_QKP + 1]
    out_ref[0, :, 0:4] = qrow[:, _QX1:_QY2 + 1] * kf
    out_ref[0, :, 4:5] = qrow[:, _QCF:_QCF + 1] * kf
    lbl_iota = jax.lax.broadcasted_iota(jnp.int32, (_NP, _C), 1).astype(f32)
    lbl = jnp.min(jnp.where(s_r == m_r, lbl_iota, float(_C)), axis=1, keepdims=True)
    out_ref[0, :, 5:6] = jnp.where(kf > 0.0, lbl, -1.0)
    out_ref[0, :, 6:7] = kf


def _run(rowpack, scores_p, interpret=False):
    B = rowpack.shape[0]
    f32 = jnp.float32
    colpack = rowpack.transpose(0, 2, 1)
    sc = scores_p.transpose(0, 2, 1)

    spec = lambda s: pl.BlockSpec(s, lambda b: (b,) + (0,) * (len(s) - 1))
    return pl.pallas_call(
        _nms_body,
        grid=(B,),
        in_specs=[
            spec((1, _NP, 5)),
            spec((1, _NP, _C)),
            spec((1, 5, _NP)),
            spec((1, _C, _NP)),
        ],
        out_specs=spec((1, _NP, 8)),
        out_shape=jax.ShapeDtypeStruct((B, _NP, 8), f32),
        compiler_params=pltpu.CompilerParams(
            dimension_semantics=("parallel",)),
        scratch_shapes=[
            pltpu.VMEM((_NP, 16), f32),
            pltpu.VMEM((16, _NP), f32),
            pltpu.VMEM((_NPP, _NP), jnp.int32),
            pltpu.VMEM((_NPP, 1), jnp.int32),
        ],
        interpret=interpret,
    )(rowpack, scores_p, colpack, sc)


def kernel(b_coords, b_o, b_scores):
    B, N, _ = b_coords.shape
    pad = _NP - N
    coords_p = jnp.pad(b_coords, ((0, 0), (0, pad), (0, 0)))
    o_p = jnp.pad(b_o, ((0, 0), (0, pad)))
    scores_p = jnp.pad(b_scores, ((0, 0), (0, pad), (0, 0)))
    rowpack = jnp.concatenate([coords_p, o_p[..., None]], axis=2)
    out = _run(rowpack, scores_p)
    boxes = out[:, :N, 0:4]
    scores = out[:, :N, 4]
    labels = out[:, :N, 5].astype(jnp.int32)
    keep = out[:, :N, 6].astype(jnp.bool_)
    return boxes, scores, labels, keep


# speedup vs baseline: 287.1898x; 1.3157x over previous
import jax
import jax.numpy as jnp
from jax.experimental import pallas as pl
from jax.experimental.pallas import tpu as pltpu

_C = 20
_NP = 5120
_T = 512
_NT = _NP // _T
_PK = 16
_TP = _T // _PK
_NPP = _NP // _PK
_IOU_T = 0.7
_SCORE_T = 0.05

_QX1, _QY1, _QX2, _QY2, _QAR, _QCF, _QID, _QVL, _QKP = range(9)


def _iou_tile(x1s, y1s, x2s, y2s, ars, x1t, y1t, x2t, y2t, art):
    ix1 = jnp.maximum(x1s, x1t)
    iy1 = jnp.maximum(y1s, y1t)
    ix2 = jnp.minimum(x2s, x2t)
    iy2 = jnp.minimum(y2s, y2t)
    iw = jnp.maximum(ix2 - ix1, 0.0)
    ih = jnp.maximum(iy2 - iy1, 0.0)
    inter = iw * ih
    union = ars + art - inter
    return inter / jnp.maximum(union, 1e-9)


def _pow2_f32(e):
    return jax.lax.bitcast_convert_type((e + 127) << 23, jnp.float32)


def _dot(a, b):
    return jax.lax.dot_general(a, b, (((1,), (0,)), ((), ())),
                               preferred_element_type=jnp.float32)


def _dot_rt(a, b):
    return jax.lax.dot_general(a, b, (((1,), (1,)), ((), ())),
                               preferred_element_type=jnp.float32)


def _nms_body(cp_ref, sc_ref, out_ref, qcol, mpa, kpr):
    f32 = jnp.float32
    i32 = jnp.int32

    c_l = jax.lax.broadcasted_iota(i32, (_TP, _T), 1)
    r_s = jax.lax.broadcasted_iota(i32, (_TP, _T), 0)
    pmat = jnp.where((c_l >> 4) == r_s, _pow2_f32(c_l & 15), 0.0)
    eye_r = jax.lax.broadcasted_iota(i32, (_T, _T), 0)
    eye_c = jax.lax.broadcasted_iota(i32, (_T, _T), 1)
    eye = jnp.where(eye_r == eye_c, 1.0, 0.0)

    cp = cp_ref[0]
    s_c = sc_ref[0]
    m_c = jnp.max(s_c, axis=0, keepdims=True)
    xc, yc, wc, hc, oc = (cp[k:k + 1, :] for k in range(5))
    x1c = jnp.clip(xc - wc / 2.0, 0.0, 1.0)
    y1c = jnp.clip(yc - hc / 2.0, 0.0, 1.0)
    x2c = jnp.clip(xc + wc / 2.0, 0.0, 1.0)
    y2c = jnp.clip(yc + hc / 2.0, 0.0, 1.0)
    areac = (x2c - x1c) * (y2c - y1c)
    conf_c = m_c * oc
    valid_c = jnp.where((oc >= 0.5) & (conf_c >= _SCORE_T), 1.0, 0.0)
    idx_c = jax.lax.broadcasted_iota(i32, (1, _NP), 1).astype(f32)
    qcol[_QX1:_QX1 + 1, :] = x1c
    qcol[_QY1:_QY1 + 1, :] = y1c
    qcol[_QX2:_QX2 + 1, :] = x2c
    qcol[_QY2:_QY2 + 1, :] = y2c
    qcol[_QAR:_QAR + 1, :] = areac
    qcol[_QCF:_QCF + 1, :] = conf_c
    qcol[_QID:_QID + 1, :] = idx_c
    qcol[_QVL:_QVL + 1, :] = valid_c
    qcol[_QKP:_QKP + 1, :] = valid_c

    def col_q(off):
        return (qcol[_QX1:_QX1 + 1, pl.ds(off, _T)],
                qcol[_QY1:_QY1 + 1, pl.ds(off, _T)],
                qcol[_QX2:_QX2 + 1, pl.ds(off, _T)],
                qcol[_QY2:_QY2 + 1, pl.ds(off, _T)],
                qcol[_QAR:_QAR + 1, pl.ds(off, _T)],
                qcol[_QCF:_QCF + 1, pl.ds(off, _T)],
                qcol[_QID:_QID + 1, pl.ds(off, _T)])

    def build_i(i, _):
        roff = pl.multiple_of(i * _T, _T)
        ci = col_q(roff)
        x1s, y1s, x2s, y2s, ars, cfs, ids = (_dot_rt(eye, q) for q in ci)
        poff = pl.multiple_of(i * _TP, _TP)

        x1t, y1t, x2t, y2t, art, cft, idt = ci
        iou = _iou_tile(x1s, y1s, x2s, y2s, ars, x1t, y1t, x2t, y2t, art)
        s_b = iou > _IOU_T
        h_f = jnp.where((cfs > cft) | ((cfs == cft) & (ids < idt)), 1.0, 0.0)
        t_a = jnp.where(s_b, h_f, 0.0)
        mpa[pl.ds(poff, _TP), pl.ds(roff, _T)] = _dot(pmat, t_a).astype(i32)

        def build_j(j, _):
            off = pl.multiple_of(j * _T, _T)
            x1t, y1t, x2t, y2t, art, cft, idt = col_q(off)
            iou = _iou_tile(x1s, y1s, x2s, y2s, ars, x1t, y1t, x2t, y2t, art)
            s_b = iou > _IOU_T
            h_f = jnp.where((cfs > cft) | ((cfs == cft) & (ids < idt)), 1.0, 0.0)
            t_a = jnp.where(s_b, h_f, 0.0)
            t_b = jnp.where(s_b, 1.0 - h_f, 0.0)
            mpa[pl.ds(poff, _TP), pl.ds(off, _T)] = _dot(pmat, t_a).astype(i32)
            mpa[pl.ds(j * _TP, _TP), pl.ds(roff, _T)] = _dot_rt(
                pmat, t_b).astype(i32)
            return 0

        jax.lax.fori_loop(i + 1, _NT, build_j, 0)
        return 0

    jax.lax.fori_loop(0, _NT, build_i, 0)

    def pack_i(i, _):
        off = pl.multiple_of(i * _T, _T)
        kpr[pl.ds(i * _TP, _TP), 0:1] = _dot_rt(
            pmat, qcol[_QVL:_QVL + 1, pl.ds(off, _T)]).astype(i32)
        return 0

    jax.lax.fori_loop(0, _NT, pack_i, 0)

    def sweep(changed_unused):
        def jbody(j, changed):
            off = pl.multiple_of(j * _T, _T)
            hit = (mpa[:, pl.ds(off, _T)] & kpr[:, 0:1]) != 0
            sup = jnp.max(jnp.where(hit, 1.0, 0.0), axis=0, keepdims=True)
            vt = qcol[_QVL:_QVL + 1, pl.ds(off, _T)]
            knew = vt * (1.0 - sup)
            old = qcol[_QKP:_QKP + 1, pl.ds(off, _T)]
            changed = changed | jnp.any(knew != old)
            qcol[_QKP:_QKP + 1, pl.ds(off, _T)] = knew
            kpr[pl.ds(j * _TP, _TP), 0:1] = _dot_rt(pmat, knew).astype(jnp.int32)
            return changed

        return jax.lax.fori_loop(0, _NT, jbody, jnp.bool_(False))

    jax.lax.while_loop(lambda c: c, sweep, jnp.bool_(True))

    kf = qcol[_QKP:_QKP + 1, :]
    out_ref[0, 0:1, :] = qcol[_QX1:_QX1 + 1, :] * kf
    out_ref[0, 1:2, :] = qcol[_QY1:_QY1 + 1, :] * kf
    out_ref[0, 2:3, :] = qcol[_QX2:_QX2 + 1, :] * kf
    out_ref[0, 3:4, :] = qcol[_QY2:_QY2 + 1, :] * kf
    out_ref[0, 4:5, :] = qcol[_QCF:_QCF + 1, :] * kf
    lbl_iota = jax.lax.broadcasted_iota(jnp.int32, (_C, _NP), 0).astype(f32)
    lbl = jnp.min(jnp.where(s_c == m_c, lbl_iota, float(_C)),
                  axis=0, keepdims=True)
    out_ref[0, 5:6, :] = jnp.where(kf > 0.0, lbl, -1.0)
    out_ref[0, 6:7, :] = kf


def _run(colpack, sc, interpret=False):
    B = colpack.shape[0]
    f32 = jnp.float32

    spec = lambda s: pl.BlockSpec(s, lambda b: (b,) + (0,) * (len(s) - 1))
    return pl.pallas_call(
        _nms_body,
        grid=(B,),
        in_specs=[
            spec((1, 5, _NP)),
            spec((1, _C, _NP)),
        ],
        out_specs=spec((1, 8, _NP)),
        out_shape=jax.ShapeDtypeStruct((B, 8, _NP), f32),
        compiler_params=pltpu.CompilerParams(
            dimension_semantics=("parallel",)),
        scratch_shapes=[
            pltpu.VMEM((16, _NP), f32),
            pltpu.VMEM((_NPP, _NP), jnp.int32),
            pltpu.VMEM((_NPP, 1), jnp.int32),
        ],
        interpret=interpret,
    )(colpack, sc)


def kernel(b_coords, b_o, b_scores):
    B, N, _ = b_coords.shape
    pad = _NP - N
    coords_p = jnp.pad(b_coords, ((0, 0), (0, pad), (0, 0)))
    o_p = jnp.pad(b_o, ((0, 0), (0, pad)))
    scores_p = jnp.pad(b_scores, ((0, 0), (0, pad), (0, 0)))
    colpack = jnp.concatenate([coords_p, o_p[..., None]], axis=2).transpose(0, 2, 1)
    out = _run(colpack, scores_p.transpose(0, 2, 1))
    outr = out.transpose(0, 2, 1)
    boxes = outr[:, :N, 0:4]
    scores = outr[:, :N, 4]
    labels = outr[:, :N, 5].astype(jnp.int32)
    keep = outr[:, :N, 6].astype(jnp.bool_)
    return boxes, scores, labels, keep
